# baseline (device time: 124521 ns/iter reference)
import jax
import jax.numpy as jnp
from jax import lax
from jax.experimental import pallas as pl
from jax.experimental.pallas import tpu as pltpu

N_DEV = 4
NEG_INF = -1e9


def kernel(x, Wq, K_ext, V_ext, Wo):
    B, Sq, Dm = x.shape
    _, Skv, Hq, Dh = K_ext.shape
    Dqk = Hq * Dh

    x2 = x.reshape(B * Sq, Dm)
    k2 = K_ext.reshape(B * Skv, Dqk)
    v2 = V_ext.reshape(B * Skv, Dqk)

    def body(x_ref, wq_ref, k_ref, v_ref, wo_ref, out_ref,
             kf_ref, vf_ref, ksend, krecv, vsend, vrecv):
        my = lax.axis_index("i")
        left = lax.rem(my + N_DEV - 1, N_DEV)
        right = lax.rem(my + 1, N_DEV)

        barrier_sem = pltpu.get_barrier_semaphore()
        for nbr in (left, right):
            pl.semaphore_signal(
                barrier_sem, inc=1,
                device_id=(nbr,), device_id_type=pl.DeviceIdType.MESH,
            )
        pl.semaphore_wait(barrier_sem, 2)

        kf_ref[0] = k_ref[...].astype(jnp.bfloat16)
        vf_ref[0] = v_ref[...].astype(jnp.bfloat16)

        for h in range(N_DEV - 1):
            rk = pltpu.make_async_remote_copy(
                src_ref=kf_ref.at[h], dst_ref=kf_ref.at[h + 1],
                send_sem=ksend.at[h], recv_sem=krecv.at[h],
                device_id=(right,), device_id_type=pl.DeviceIdType.MESH,
            )
            rv = pltpu.make_async_remote_copy(
                src_ref=vf_ref.at[h], dst_ref=vf_ref.at[h + 1],
                send_sem=vsend.at[h], recv_sem=vrecv.at[h],
                device_id=(right,), device_id_type=pl.DeviceIdType.MESH,
            )
            rk.start()
            rv.start()
            if h == 0:
                q = lax.dot_general(
                    x_ref[...].astype(jnp.bfloat16),
                    wq_ref[...].astype(jnp.bfloat16),
                    (((1,), (0,)), ((), ())),
                    preferred_element_type=jnp.float32,
                )
                q = (q * 0.125).astype(jnp.bfloat16)
            rk.wait()
            rv.wait()

        masks = []
        for s in range(N_DEV):
            origin = lax.rem(my - s + N_DEV, N_DEV)
            qi = my * Sq + lax.broadcasted_iota(jnp.int32, (Sq, Skv), 0)
            kj = origin * Skv + lax.broadcasted_iota(jnp.int32, (Sq, Skv), 1)
            masks.append(
                (jnp.abs(qi - kj) <= 128) | (kj < 32) | (qi < 32)
            )

        wo_b = wo_ref[...].astype(jnp.bfloat16)

        for b in range(B):
            ctx_heads = []
            for h in range(Hq):
                qbh = q[b * Sq:(b + 1) * Sq, h * Dh:(h + 1) * Dh]
                scores = []
                for s in range(N_DEV):
                    kslot = kf_ref[s, b * Skv:(b + 1) * Skv,
                                   h * Dh:(h + 1) * Dh]
                    sc = lax.dot_general(
                        qbh, kslot, (((1,), (1,)), ((), ())),
                        preferred_element_type=jnp.float32,
                    )
                    scores.append(jnp.where(masks[s], sc, NEG_INF))
                sc_all = jnp.concatenate(scores, axis=1)
                m = jnp.max(sc_all, axis=1, keepdims=True)
                w = jnp.exp(sc_all - m)
                l = jnp.sum(w, axis=1, keepdims=True)
                wn = (w / l).astype(jnp.bfloat16)
                acc = jnp.zeros((Sq, Dh), jnp.float32)
                for s in range(N_DEV):
                    vslot = vf_ref[s, b * Skv:(b + 1) * Skv,
                                   h * Dh:(h + 1) * Dh]
                    acc += lax.dot_general(
                        wn[:, s * Skv:(s + 1) * Skv], vslot,
                        (((1,), (0,)), ((), ())),
                        preferred_element_type=jnp.float32,
                    )
                ctx_heads.append(acc.astype(jnp.bfloat16))
            ctx_b = jnp.concatenate(ctx_heads, axis=1)
            out_ref[b * Sq:(b + 1) * Sq, :] = lax.dot_general(
                ctx_b, wo_b, (((1,), (0,)), ((), ())),
                preferred_element_type=jnp.float32,
            )

    out2 = pl.pallas_call(
        body,
        out_shape=jax.ShapeDtypeStruct((B * Sq, Dm), jnp.float32),
        in_specs=[pl.BlockSpec(memory_space=pltpu.VMEM)] * 5,
        out_specs=pl.BlockSpec(memory_space=pltpu.VMEM),
        scratch_shapes=[
            pltpu.VMEM((N_DEV, B * Skv, Dqk), jnp.bfloat16),
            pltpu.VMEM((N_DEV, B * Skv, Dqk), jnp.bfloat16),
            pltpu.SemaphoreType.DMA((N_DEV - 1,)),
            pltpu.SemaphoreType.DMA((N_DEV - 1,)),
            pltpu.SemaphoreType.DMA((N_DEV - 1,)),
            pltpu.SemaphoreType.DMA((N_DEV - 1,)),
        ],
        compiler_params=pltpu.CompilerParams(collective_id=0),
    )(x2, Wq, k2, v2, Wo)

    return out2.reshape(B, Sq, Dm)


# device time: 66634 ns/iter; 1.8687x vs baseline; 1.8687x over previous
import jax
import jax.numpy as jnp
from jax import lax
from jax.experimental import pallas as pl
from jax.experimental.pallas import tpu as pltpu

N_DEV = 4
NEG_INF = -1e9


def kernel(x, Wq, K_ext, V_ext, Wo):
    B, Sq, Dm = x.shape
    _, Skv, Hq, Dh = K_ext.shape
    Dqk = Hq * Dh
    half = B * Skv // 2

    x2 = x.reshape(B * Sq, Dm)
    k2 = K_ext.reshape(B * Skv, Dqk)
    v2 = V_ext.reshape(B * Skv, Dqk)

    def body(x_ref, wq_ref, k_ref, v_ref, wo_ref, out_ref,
             kf_ref, vf_ref, ksend, krecv, vsend, vrecv):
        my = lax.axis_index("i")
        left = lax.rem(my + N_DEV - 1, N_DEV)
        right = lax.rem(my + 1, N_DEV)

        barrier_sem = pltpu.get_barrier_semaphore()
        for nbr in (left, right):
            pl.semaphore_signal(
                barrier_sem, inc=1,
                device_id=(nbr,), device_id_type=pl.DeviceIdType.MESH,
            )
        pl.semaphore_wait(barrier_sem, 2)

        kf_ref[0] = k_ref[...].astype(jnp.bfloat16)
        vf_ref[0] = v_ref[...].astype(jnp.bfloat16)

        def copy(ref, sems_s, sems_r, idx, src, dst, target):
            return pltpu.make_async_remote_copy(
                src_ref=src, dst_ref=dst,
                send_sem=sems_s.at[idx], recv_sem=sems_r.at[idx],
                device_id=(target,), device_id_type=pl.DeviceIdType.MESH,
            )

        rk_r = copy(kf_ref, ksend, krecv, 0, kf_ref.at[0], kf_ref.at[1], right)
        rv_r = copy(vf_ref, vsend, vrecv, 0, vf_ref.at[0], vf_ref.at[1], right)
        rk_l = copy(kf_ref, ksend, krecv, 1, kf_ref.at[0], kf_ref.at[2], left)
        rv_l = copy(vf_ref, vsend, vrecv, 1, vf_ref.at[0], vf_ref.at[2], left)
        rk_r.start()
        rv_r.start()
        rk_l.start()
        rv_l.start()

        q = lax.dot_general(
            x_ref[...].astype(jnp.bfloat16),
            wq_ref[...].astype(jnp.bfloat16),
            (((1,), (0,)), ((), ())),
            preferred_element_type=jnp.float32,
        )
        q = (q * 0.125).astype(jnp.bfloat16)

        iota_q = lax.broadcasted_iota(jnp.int32, (Sq, Skv), 0)
        iota_k = lax.broadcasted_iota(jnp.int32, (Sq, Skv), 1)

        def process_slot(slot, origin, state):
            qi = my * Sq + iota_q
            kj = origin * Skv + iota_k
            mask = (jnp.abs(qi - kj) <= 128) | (kj < 32) | (qi < 32)
            new_state = {}
            for b in range(B):
                for h in range(Hq):
                    qbh = q[b * Sq:(b + 1) * Sq, h * Dh:(h + 1) * Dh]
                    kslot = kf_ref[slot, b * Skv:(b + 1) * Skv,
                                   h * Dh:(h + 1) * Dh]
                    sc = lax.dot_general(
                        qbh, kslot, (((1,), (1,)), ((), ())),
                        preferred_element_type=jnp.float32,
                    )
                    w = jnp.exp(jnp.where(mask, sc, NEG_INF))
                    vslot = vf_ref[slot, b * Skv:(b + 1) * Skv,
                                   h * Dh:(h + 1) * Dh]
                    l_c = jnp.sum(w, axis=1, keepdims=True)
                    acc_c = lax.dot_general(
                        w.astype(jnp.bfloat16), vslot,
                        (((1,), (0,)), ((), ())),
                        preferred_element_type=jnp.float32,
                    )
                    if state is None:
                        new_state[b, h] = (l_c, acc_c)
                    else:
                        l0, acc0 = state[b, h]
                        new_state[b, h] = (l0 + l_c, acc0 + acc_c)
            return new_state

        state = process_slot(0, my, None)

        rk_r.wait_recv()
        fk_r = copy(kf_ref, ksend, krecv, 2,
                    kf_ref.at[1, pl.ds(0, half)],
                    kf_ref.at[3, pl.ds(0, half)], right)
        fk_r.start()
        rv_r.wait_recv()
        fv_r = copy(vf_ref, vsend, vrecv, 2,
                    vf_ref.at[1, pl.ds(0, half)],
                    vf_ref.at[3, pl.ds(0, half)], right)
        fv_r.start()
        rk_l.wait_recv()
        fk_l = copy(kf_ref, ksend, krecv, 3,
                    kf_ref.at[2, pl.ds(half, half)],
                    kf_ref.at[3, pl.ds(half, half)], left)
        fk_l.start()
        rv_l.wait_recv()
        fv_l = copy(vf_ref, vsend, vrecv, 3,
                    vf_ref.at[2, pl.ds(half, half)],
                    vf_ref.at[3, pl.ds(half, half)], left)
        fv_l.start()

        state = process_slot(1, left, state)
        state = process_slot(2, right, state)

        fk_r.wait_recv()
        fv_r.wait_recv()
        fk_l.wait_recv()
        fv_l.wait_recv()
        state = process_slot(3, lax.rem(my + 2, N_DEV), state)

        wo_b = wo_ref[...].astype(jnp.bfloat16)
        for b in range(B):
            ctx_b = jnp.concatenate(
                [(state[b, h][1] / state[b, h][0]).astype(jnp.bfloat16)
                 for h in range(Hq)], axis=1)
            out_ref[b * Sq:(b + 1) * Sq, :] = lax.dot_general(
                ctx_b, wo_b, (((1,), (0,)), ((), ())),
                preferred_element_type=jnp.float32,
            )

        for d in (rk_r, rv_r, rk_l, rv_l, fk_r, fv_r, fk_l, fv_l):
            d.wait_send()

    out2 = pl.pallas_call(
        body,
        out_shape=jax.ShapeDtypeStruct((B * Sq, Dm), jnp.float32),
        in_specs=[pl.BlockSpec(memory_space=pltpu.VMEM)] * 5,
        out_specs=pl.BlockSpec(memory_space=pltpu.VMEM),
        scratch_shapes=[
            pltpu.VMEM((N_DEV, B * Skv, Dqk), jnp.bfloat16),
            pltpu.VMEM((N_DEV, B * Skv, Dqk), jnp.bfloat16),
            pltpu.SemaphoreType.DMA((4,)),
            pltpu.SemaphoreType.DMA((4,)),
            pltpu.SemaphoreType.DMA((4,)),
            pltpu.SemaphoreType.DMA((4,)),
        ],
        compiler_params=pltpu.CompilerParams(
            collective_id=0, vmem_limit_bytes=100 * 1024 * 1024,
        ),
    )(x2, Wq, k2, v2, Wo)

    return out2.reshape(B, Sq, Dm)


# device time: 65960 ns/iter; 1.8878x vs baseline; 1.0102x over previous
import jax
import jax.numpy as jnp
from jax import lax
from jax.experimental import pallas as pl
from jax.experimental.pallas import tpu as pltpu

N_DEV = 4
NEG_INF = -1e9
BAND = 128
NGLOB = 32


def kernel(x, Wq, K_ext, V_ext, Wo):
    B, Sq, Dm = x.shape
    _, Skv, Hq, Dh = K_ext.shape
    Dqk = Hq * Dh
    half = B * Skv // 2

    x2 = x.reshape(B * Sq, Dm)
    k2 = K_ext.reshape(B * Skv, Dqk)
    v2 = V_ext.reshape(B * Skv, Dqk)

    def body(x_ref, wq_ref, k_ref, v_ref, wo_ref, out_ref,
             kf_ref, vf_ref, ksend, krecv, vsend, vrecv):
        my = lax.axis_index("i")
        left = lax.rem(my + N_DEV - 1, N_DEV)
        right = lax.rem(my + 1, N_DEV)
        origin = [my, left, right, lax.rem(my + 2, N_DEV)]

        barrier_sem = pltpu.get_barrier_semaphore()
        for nbr in (left, right):
            pl.semaphore_signal(
                barrier_sem, inc=1,
                device_id=(nbr,), device_id_type=pl.DeviceIdType.MESH,
            )
        pl.semaphore_wait(barrier_sem, 2)

        kf_ref[0] = k_ref[...].astype(jnp.bfloat16)
        vf_ref[0] = v_ref[...].astype(jnp.bfloat16)

        def copy(sems_s, sems_r, idx, src, dst, target):
            return pltpu.make_async_remote_copy(
                src_ref=src, dst_ref=dst,
                send_sem=sems_s.at[idx], recv_sem=sems_r.at[idx],
                device_id=(target,), device_id_type=pl.DeviceIdType.MESH,
            )

        rk_r = copy(ksend, krecv, 0, kf_ref.at[0], kf_ref.at[1], right)
        rv_r = copy(vsend, vrecv, 0, vf_ref.at[0], vf_ref.at[1], right)
        rk_l = copy(ksend, krecv, 1, kf_ref.at[0], kf_ref.at[2], left)
        rv_l = copy(vsend, vrecv, 1, vf_ref.at[0], vf_ref.at[2], left)
        rk_r.start()
        rv_r.start()
        rk_l.start()
        rv_l.start()

        q = lax.dot_general(
            x_ref[...].astype(jnp.bfloat16),
            wq_ref[...].astype(jnp.bfloat16),
            (((1,), (0,)), ((), ())),
            preferred_element_type=jnp.float32,
        )
        q = (q * 0.125).astype(jnp.bfloat16)

        is_root = (my == 0).astype(jnp.float32)

        def qblk(b, h, r0=0, nrows=Sq):
            return q[b * Sq + r0:b * Sq + r0 + nrows,
                     h * Dh:(h + 1) * Dh]

        def kvblk(ref, slot, b, h, r0, nrows):
            return ref[slot, b * Skv + r0:b * Skv + r0 + nrows,
                       h * Dh:(h + 1) * Dh]

        def scores(qb, kb, mask=None, sel=None):
            sc = lax.dot_general(
                qb, kb, (((1,), (1,)), ((), ())),
                preferred_element_type=jnp.float32,
            )
            if mask is not None:
                sc = jnp.where(mask, sc, NEG_INF)
            w = jnp.exp(sc)
            if sel is not None:
                w = w * sel
            return w

        def pv(w, vb):
            return lax.dot_general(
                w.astype(jnp.bfloat16), vb, (((1,), (0,)), ((), ())),
                preferred_element_type=jnp.float32,
            )

        def accumulate(state, key, w, vb):
            l_c = jnp.sum(w, axis=1, keepdims=True)
            acc_c = pv(w, vb)
            if key in state:
                l0, acc0 = state[key]
                state[key] = (l0 + l_c, acc0 + acc_c)
            else:
                state[key] = (l_c, acc_c)

        iq = lax.broadcasted_iota(jnp.int32, (Sq, Skv), 0)
        ik = lax.broadcasted_iota(jnp.int32, (Sq, Skv), 1)
        qi_own = my * Sq + iq
        kj_own = my * Skv + ik
        mask_own = ((jnp.abs(qi_own - kj_own) <= BAND)
                    | (kj_own < NGLOB) | (qi_own < NGLOB))
        state = {}
        gstate = {}
        for b in range(B):
            for h in range(Hq):
                w = scores(qblk(b, h), kvblk(kf_ref, 0, b, h, 0, Skv),
                           mask=mask_own)
                accumulate(state, (b, h), w, kvblk(vf_ref, 0, b, h, 0, Skv))

        rk_r.wait_recv()
        fk_r = copy(ksend, krecv, 2,
                    kf_ref.at[1, pl.ds(0, half)],
                    kf_ref.at[3, pl.ds(0, half)], right)
        fk_r.start()
        rv_r.wait_recv()
        fv_r = copy(vsend, vrecv, 2,
                    vf_ref.at[1, pl.ds(0, half)],
                    vf_ref.at[3, pl.ds(0, half)], right)
        fv_r.start()
        rk_l.wait_recv()
        fk_l = copy(ksend, krecv, 3,
                    kf_ref.at[2, pl.ds(half, half)],
                    kf_ref.at[3, pl.ds(half, half)], left)
        fk_l.start()
        rv_l.wait_recv()
        fv_l = copy(vsend, vrecv, 3,
                    vf_ref.at[2, pl.ds(half, half)],
                    vf_ref.at[3, pl.ds(half, half)], left)
        fv_l.start()

        iqh = lax.broadcasted_iota(jnp.int32, (Sq, BAND), 0)
        ikh = lax.broadcasted_iota(jnp.int32, (Sq, BAND), 1)
        qi_h = my * Sq + iqh
        kj_p2 = origin[1] * Skv + (Skv - BAND) + ikh
        mask_p2 = jnp.abs(qi_h - kj_p2) <= BAND
        kj_p3 = origin[2] * Skv + ikh
        mask_p3 = jnp.abs(qi_h - kj_p3) <= BAND

        def halo_piece(slot, r0, mask):
            for b in range(B):
                for h in range(Hq):
                    w = scores(qblk(b, h),
                               kvblk(kf_ref, slot, b, h, r0, BAND),
                               mask=mask)
                    accumulate(state, (b, h), w,
                               kvblk(vf_ref, slot, b, h, r0, BAND))

        def globq_piece(slot):
            for b in range(B):
                for h in range(Hq):
                    w = scores(qblk(b, h, 0, NGLOB),
                               kvblk(kf_ref, slot, b, h, 0, Skv),
                               sel=is_root)
                    accumulate(gstate, (b, h), w,
                               kvblk(vf_ref, slot, b, h, 0, Skv))

        def globk_piece(slot):
            sel = ((origin[slot] == 0) & (my != 0)).astype(jnp.float32)
            for b in range(B):
                for h in range(Hq):
                    w = scores(qblk(b, h),
                               kvblk(kf_ref, slot, b, h, 0, NGLOB),
                               sel=sel)
                    accumulate(state, (b, h), w,
                               kvblk(vf_ref, slot, b, h, 0, NGLOB))

        halo_piece(1, Skv - BAND, mask_p2)
        halo_piece(2, 0, mask_p3)
        globq_piece(1)
        globq_piece(2)
        globk_piece(1)
        globk_piece(2)

        fk_r.wait_recv()
        fv_r.wait_recv()
        fk_l.wait_recv()
        fv_l.wait_recv()
        globq_piece(3)
        globk_piece(3)

        wo_b = wo_ref[...].astype(jnp.bfloat16)
        for b in range(B):
            cols = []
            for h in range(Hq):
                l, acc = state[(b, h)]
                gl, gacc = gstate[(b, h)]
                top = ((acc[0:NGLOB] + gacc)
                       / (l[0:NGLOB] + gl)).astype(jnp.bfloat16)
                rest = (acc[NGLOB:] / l[NGLOB:]).astype(jnp.bfloat16)
                cols.append(jnp.concatenate([top, rest], axis=0))
            ctx_b = jnp.concatenate(cols, axis=1)
            out_ref[b * Sq:(b + 1) * Sq, :] = lax.dot_general(
                ctx_b, wo_b, (((1,), (0,)), ((), ())),
                preferred_element_type=jnp.float32,
            )

        for d in (rk_r, rv_r, rk_l, rv_l, fk_r, fv_r, fk_l, fv_l):
            d.wait_send()

    out2 = pl.pallas_call(
        body,
        out_shape=jax.ShapeDtypeStruct((B * Sq, Dm), jnp.float32),
        in_specs=[pl.BlockSpec(memory_space=pltpu.VMEM)] * 5,
        out_specs=pl.BlockSpec(memory_space=pltpu.VMEM),
        scratch_shapes=[
            pltpu.VMEM((N_DEV, B * Skv, Dqk), jnp.bfloat16),
            pltpu.VMEM((N_DEV, B * Skv, Dqk), jnp.bfloat16),
            pltpu.SemaphoreType.DMA((4,)),
            pltpu.SemaphoreType.DMA((4,)),
            pltpu.SemaphoreType.DMA((4,)),
            pltpu.SemaphoreType.DMA((4,)),
        ],
        compiler_params=pltpu.CompilerParams(
            collective_id=0, vmem_limit_bytes=100 * 1024 * 1024,
        ),
    )(x2, Wq, k2, v2, Wo)

    return out2.reshape(B, Sq, Dm)


# device time: 45672 ns/iter; 2.7264x vs baseline; 1.4442x over previous
import functools

import jax
import jax.numpy as jnp
from jax import lax
from jax.experimental import pallas as pl
from jax.experimental.pallas import tpu as pltpu

N_DEV = 4
NEG_INF = -1e9
BAND = 128
NGLOB = 32


def kernel(x, Wq, K_ext, V_ext, Wo):
    B, Sq, Dm = x.shape
    _, Skv, Hq, Dh = K_ext.shape
    Dqk = Hq * Dh

    x2 = x.reshape(B * Sq, Dm)
    k2 = K_ext.reshape(B * Skv, Dqk)
    v2 = V_ext.reshape(B * Skv, Dqk)

    def body(x_ref, wq_ref, k_ref, v_ref, wo_ref, out_ref,
             ho_r, ho_l, hi_l, hi_r, gk_out, gk_in, qg_out, qg_in,
             part_out, part_in,
             hsend, hrecv, qgsend, qgrecv, gksend, gkrecv, psend, precv):
        my = lax.axis_index("i")
        left = lax.rem(my + N_DEV - 1, N_DEV)
        right = lax.rem(my + 1, N_DEV)
        is_root = my == 0

        gk_in[...] = jnp.zeros((2 * B * NGLOB, Dqk), jnp.bfloat16)

        barrier_sem = pltpu.get_barrier_semaphore()
        for nbr in (left, right):
            pl.semaphore_signal(
                barrier_sem, inc=1,
                device_id=(nbr,), device_id_type=pl.DeviceIdType.MESH,
            )
        pl.semaphore_wait(barrier_sem, 2)

        kown = k_ref[...].astype(jnp.bfloat16)
        vown = v_ref[...].astype(jnp.bfloat16)

        for b in range(B):
            ho_r[b * BAND:(b + 1) * BAND] = (
                kown[b * Skv + Skv - BAND:(b + 1) * Skv])
            ho_r[(B + b) * BAND:(B + b + 1) * BAND] = (
                vown[b * Skv + Skv - BAND:(b + 1) * Skv])
            ho_l[b * BAND:(b + 1) * BAND] = (
                kown[b * Skv:b * Skv + BAND])
            ho_l[(B + b) * BAND:(B + b + 1) * BAND] = (
                vown[b * Skv:b * Skv + BAND])

        def copy(src, dst, ssem, rsem, target):
            return pltpu.make_async_remote_copy(
                src_ref=src, dst_ref=dst, send_sem=ssem, recv_sem=rsem,
                device_id=(target,), device_id_type=pl.DeviceIdType.MESH,
            )

        h_r = copy(ho_r, hi_l, hsend.at[0], hrecv.at[0], right)
        h_l = copy(ho_l, hi_r, hsend.at[1], hrecv.at[1], left)
        h_r.start()
        h_l.start()

        for b in range(B):
            gk_out[b * NGLOB:(b + 1) * NGLOB] = (
                kown[b * Skv:b * Skv + NGLOB])
            gk_out[(B + b) * NGLOB:(B + b + 1) * NGLOB] = (
                vown[b * Skv:b * Skv + NGLOB])

        q = lax.dot_general(
            x_ref[...].astype(jnp.bfloat16),
            wq_ref[...].astype(jnp.bfloat16),
            (((1,), (0,)), ((), ())),
            preferred_element_type=jnp.float32,
        )
        q = (q * 0.125).astype(jnp.bfloat16)

        qg_local = jnp.concatenate(
            [q[0:NGLOB], q[Sq:Sq + NGLOB]], axis=0)
        qg_out[...] = qg_local

        qg_d = [copy(qg_out, qg_in, qgsend.at[j], qgrecv.at[0], t)
                for j, t in enumerate((1, 2, 3))]
        gk_d = [copy(gk_out, gk_in, gksend.at[j], gkrecv.at[0], t)
                for j, t in enumerate((1, 2, 3))]

        @pl.when(is_root)
        def _():
            for d in qg_d + gk_d:
                d.start()

        @pl.when(jnp.logical_not(is_root))
        def _():
            qg_d[0].wait_recv()

        qg_val = jnp.where(is_root, qg_local, qg_in[...])
        ones = jnp.ones((Skv, Dh), jnp.bfloat16)
        for b in range(B):
            for h in range(Hq):
                qgb = qg_val[b * NGLOB:(b + 1) * NGLOB,
                             h * Dh:(h + 1) * Dh]
                kb = kown[b * Skv:(b + 1) * Skv, h * Dh:(h + 1) * Dh]
                sc = lax.dot_general(
                    qgb, kb, (((1,), (1,)), ((), ())),
                    preferred_element_type=jnp.float32,
                )
                w = jnp.exp(sc)
                vext = jnp.concatenate(
                    [vown[b * Skv:(b + 1) * Skv, h * Dh:(h + 1) * Dh],
                     ones], axis=1)
                pe = lax.dot_general(
                    w.astype(jnp.bfloat16), vext, (((1,), (0,)), ((), ())),
                    preferred_element_type=jnp.float32,
                )
                r0 = (b * Hq + h) * NGLOB
                part_out[r0:r0 + NGLOB] = pe.astype(jnp.bfloat16)

        pd = [copy(part_out, part_in.at[s], psend.at[0], precv.at[s], 0)
              for s in range(3)]
        for s in range(3):
            @pl.when(my == s + 1)
            def _(s=s):
                pd[s].start()

        iq = lax.broadcasted_iota(jnp.int32, (Sq, Skv), 0)
        ik = lax.broadcasted_iota(jnp.int32, (Sq, Skv), 1)
        qi_own = my * Sq + iq
        kj_own = my * Skv + ik
        mask_own = (jnp.abs(qi_own - kj_own) <= BAND) | (kj_own < NGLOB)

        def scores(qb, kb, mask):
            sc = lax.dot_general(
                qb, kb, (((1,), (1,)), ((), ())),
                preferred_element_type=jnp.float32,
            )
            return jnp.exp(jnp.where(mask, sc, NEG_INF))

        def accumulate(state, key, w, vb):
            l_c = jnp.sum(w, axis=1, keepdims=True)
            acc_c = lax.dot_general(
                w.astype(jnp.bfloat16), vb, (((1,), (0,)), ((), ())),
                preferred_element_type=jnp.float32,
            )
            if key in state:
                l0, acc0 = state[key]
                state[key] = (l0 + l_c, acc0 + acc_c)
            else:
                state[key] = (l_c, acc_c)

        state = {}
        for b in range(B):
            for h in range(Hq):
                w = scores(q[b * Sq:(b + 1) * Sq, h * Dh:(h + 1) * Dh],
                           kown[b * Skv:(b + 1) * Skv, h * Dh:(h + 1) * Dh],
                           mask_own)
                accumulate(state, (b, h), w,
                           vown[b * Skv:(b + 1) * Skv, h * Dh:(h + 1) * Dh])

        h_r.wait_recv()
        h_l.wait_recv()
        iqh = lax.broadcasted_iota(jnp.int32, (Sq, BAND), 0)
        ikh = lax.broadcasted_iota(jnp.int32, (Sq, BAND), 1)
        qi_h = my * Sq + iqh
        mask_p2 = jnp.abs(qi_h - (left * Skv + Skv - BAND + ikh)) <= BAND
        mask_p3 = jnp.abs(qi_h - (right * Skv + ikh)) <= BAND
        for hb, mask in ((hi_l, mask_p2), (hi_r, mask_p3)):
            for b in range(B):
                for h in range(Hq):
                    w = scores(
                        q[b * Sq:(b + 1) * Sq, h * Dh:(h + 1) * Dh],
                        hb[b * BAND:(b + 1) * BAND, h * Dh:(h + 1) * Dh],
                        mask)
                    accumulate(state, (b, h), w,
                               hb[(B + b) * BAND:(B + b + 1) * BAND,
                                  h * Dh:(h + 1) * Dh])

        @pl.when(jnp.logical_not(is_root))
        def _():
            gk_d[0].wait_recv()

        not_root = jnp.logical_not(is_root)
        for b in range(B):
            for h in range(Hq):
                w = scores(q[b * Sq:(b + 1) * Sq, h * Dh:(h + 1) * Dh],
                           gk_in[b * NGLOB:(b + 1) * NGLOB,
                                 h * Dh:(h + 1) * Dh],
                           not_root)
                accumulate(state, (b, h), w,
                           gk_in[(B + b) * NGLOB:(B + b + 1) * NGLOB,
                                 h * Dh:(h + 1) * Dh])

        @pl.when(is_root)
        def _():
            for s in range(3):
                pd[s].wait_recv()

        total = part_out[...].astype(jnp.float32)
        for s in range(3):
            total = total + part_in[s].astype(jnp.float32)

        wo_b = wo_ref[...].astype(jnp.bfloat16)
        for b in range(B):
            cols = []
            for h in range(Hq):
                l, acc = state[(b, h)]
                r0 = (b * Hq + h) * NGLOB
                gacc = total[r0:r0 + NGLOB, 0:Dh]
                gl = total[r0:r0 + NGLOB, Dh:Dh + 1]
                top = jnp.where(is_root, gacc / gl,
                                acc[0:NGLOB] / l[0:NGLOB])
                rest = acc[NGLOB:] / l[NGLOB:]
                cols.append(jnp.concatenate(
                    [top.astype(jnp.bfloat16), rest.astype(jnp.bfloat16)],
                    axis=0))
            ctx_b = jnp.concatenate(cols, axis=1)
            out_ref[b * Sq:(b + 1) * Sq, :] = lax.dot_general(
                ctx_b, wo_b, (((1,), (0,)), ((), ())),
                preferred_element_type=jnp.float32,
            )

        h_r.wait_send()
        h_l.wait_send()

        @pl.when(is_root)
        def _():
            for d in qg_d + gk_d:
                d.wait_send()

        for s in range(3):
            @pl.when(my == s + 1)
            def _(s=s):
                pd[s].wait_send()

        @functools.partial(pl.run_scoped, sem2=pltpu.SemaphoreType.REGULAR)
        def _(sem2):
            for nbr in (left, right):
                pl.semaphore_signal(
                    sem2, inc=1,
                    device_id=(nbr,), device_id_type=pl.DeviceIdType.MESH,
                )
            pl.semaphore_wait(sem2, 2)

    out2 = pl.pallas_call(
        body,
        out_shape=jax.ShapeDtypeStruct((B * Sq, Dm), jnp.float32),
        in_specs=[pl.BlockSpec(memory_space=pltpu.VMEM)] * 5,
        out_specs=pl.BlockSpec(memory_space=pltpu.VMEM),
        scratch_shapes=[
            pltpu.VMEM((2 * B * BAND, Dqk), jnp.bfloat16),
            pltpu.VMEM((2 * B * BAND, Dqk), jnp.bfloat16),
            pltpu.VMEM((2 * B * BAND, Dqk), jnp.bfloat16),
            pltpu.VMEM((2 * B * BAND, Dqk), jnp.bfloat16),
            pltpu.VMEM((2 * B * NGLOB, Dqk), jnp.bfloat16),
            pltpu.VMEM((2 * B * NGLOB, Dqk), jnp.bfloat16),
            pltpu.VMEM((B * NGLOB, Dqk), jnp.bfloat16),
            pltpu.VMEM((B * NGLOB, Dqk), jnp.bfloat16),
            pltpu.VMEM((B * Hq * NGLOB, 2 * Dh), jnp.bfloat16),
            pltpu.VMEM((3, B * Hq * NGLOB, 2 * Dh), jnp.bfloat16),
            pltpu.SemaphoreType.DMA((2,)),
            pltpu.SemaphoreType.DMA((2,)),
            pltpu.SemaphoreType.DMA((3,)),
            pltpu.SemaphoreType.DMA((1,)),
            pltpu.SemaphoreType.DMA((3,)),
            pltpu.SemaphoreType.DMA((1,)),
            pltpu.SemaphoreType.DMA((1,)),
            pltpu.SemaphoreType.DMA((3,)),
        ],
        compiler_params=pltpu.CompilerParams(
            collective_id=0, vmem_limit_bytes=100 * 1024 * 1024,
        ),
    )(x2, Wq, k2, v2, Wo)

    return out2.reshape(B, Sq, Dm)
